# baseline (speedup 1.0000x reference)
import functools
import math

import jax
import jax.numpy as jnp
from jax import lax
from jax.experimental import pallas as pl
from jax.experimental.pallas import tpu as pltpu
from jax.experimental.pallas import tpu_sc as plsc

M = 0.5
COS_M = math.cos(M)
SIN_M = math.sin(M)

ROW_BLOCK = 16
COL_BLOCK = 100000

_NC = 2
_NS = 16
_NW = _NC * _NS
_N_ROWS = 1024
_RPW = _N_ROWS // _NW
_WIN = 16
_N_COLS = 100000
_TILE = 128
_MAX_CTILE = _N_COLS // _TILE - 1


def _sc_gather_body(cosine_hbm, label_hbm, fix_hbm, lab_v, win_v, fix_v, sem):
    wid = lax.axis_index("s") * _NC + lax.axis_index("c")
    base = wid * _RPW
    pltpu.sync_copy(label_hbm.at[pl.ds(base, _RPW)], lab_v)

    labs = [lab_v[pl.ds(g * 16, 16)] for g in range(_RPW // 16)]

    def _lab(r):
        return labs[r // 16][r % 16]

    def _ctile(l):
        return jnp.minimum(jnp.maximum(l >> 7, 0), _MAX_CTILE)

    copies = []
    for r in range(_RPW):
        l = _lab(r)
        c0 = pl.multiple_of(_ctile(l) * _TILE, _TILE)
        row0 = pl.multiple_of(base + (r // 8) * 8, 8)
        cp = pltpu.make_async_copy(
            cosine_hbm.at[pl.ds(row0, 8), pl.ds(c0, _TILE)],
            win_v.at[r],
            sem,
        )
        cp.start()
        copies.append(cp)
    for cp in copies:
        cp.wait()

    iot = lax.iota(jnp.int32, _WIN)
    for g in range(_RPW // 16):
        lv = labs[g]
        ct = jnp.minimum(jnp.maximum(lv >> 7, 0), _MAX_CTILE)
        off = jnp.minimum(lv - ct * _TILE, _TILE - 1)
        ridx = g * 16 + iot
        fix_v[pl.ds(g * 16, _WIN)] = plsc.load_gather(
            win_v, [ridx, ridx & 7, off]
        )
    pltpu.sync_copy(fix_v, fix_hbm.at[pl.ds(base, _RPW)])


_sc_gather = functools.partial(
    pl.kernel,
    mesh=plsc.VectorSubcoreMesh(core_axis_name="c", subcore_axis_name="s"),
    out_type=jax.ShapeDtypeStruct((_N_ROWS,), jnp.float32),
    compiler_params=pltpu.CompilerParams(needs_layout_passes=False),
    scratch_types=[
        pltpu.VMEM((_RPW,), jnp.int32),
        pltpu.VMEM((_RPW, 8, _TILE), jnp.float32),
        pltpu.VMEM((_RPW,), jnp.float32),
        pltpu.SemaphoreType.DMA,
    ],
)(_sc_gather_body)


def _dense_body(label_ref, fixv_ref, scale_ref, cosine_ref, cosine2_ref, out_ref):
    x = cosine_ref[...]
    x2 = cosine2_ref[...]
    s = scale_ref[0]
    out_ref[...] = x * s + x2 * (s * 1e-30)


@jax.jit
def kernel(cosine, label, scale):
    n_rows, n_cols = cosine.shape
    fixv = jnp.zeros((n_rows,), jnp.float32)
    grid = (n_rows // ROW_BLOCK, pl.cdiv(n_cols, COL_BLOCK))
    return pl.pallas_call(
        _dense_body,
        grid=grid,
        in_specs=[
            pl.BlockSpec((n_rows,), lambda i, j: (0,)),
            pl.BlockSpec((n_rows,), lambda i, j: (0,)),
            pl.BlockSpec((1,), lambda i, j: (0,)),
            pl.BlockSpec((ROW_BLOCK, COL_BLOCK), lambda i, j: (i, j)),
            pl.BlockSpec((ROW_BLOCK, COL_BLOCK),
                         lambda i, j, g=n_rows // ROW_BLOCK: (g - 1 - i, j)),
        ],
        out_specs=pl.BlockSpec((ROW_BLOCK, COL_BLOCK), lambda i, j: (i, j)),
        out_shape=jax.ShapeDtypeStruct(cosine.shape, cosine.dtype),
    )(label, fixv, scale, cosine, cosine)


# speedup vs baseline: 1.1015x; 1.1015x over previous
import functools
import math

import jax
import jax.numpy as jnp
from jax import lax
from jax.experimental import pallas as pl
from jax.experimental.pallas import tpu as pltpu
from jax.experimental.pallas import tpu_sc as plsc

M = 0.5
COS_M = math.cos(M)
SIN_M = math.sin(M)

ROW_BLOCK = 16
COL_BLOCK = 100000

_NC = 2
_NS = 16
_NW = _NC * _NS
_N_ROWS = 1024
_RPW = _N_ROWS // _NW
_WIN = 16
_N_COLS = 100000
_TILE = 128
_MAX_CTILE = (_N_COLS - 1) // _TILE


def _sc_gather_body(cosine_hbm, label_hbm, fix_hbm, lab_v, win_v, fix_v, sem):
    wid = lax.axis_index("s") * _NC + lax.axis_index("c")
    base = wid * _RPW
    pltpu.sync_copy(label_hbm.at[pl.ds(base, _RPW)], lab_v)

    labs = [lab_v[pl.ds(g * 16, 16)] for g in range(_RPW // 16)]

    def _lab(r):
        return labs[r // 16][r % 16]

    def _ctile(l):
        return jnp.minimum(jnp.maximum(l >> 7, 0), _MAX_CTILE)

    copies = []
    for r in range(_RPW):
        l = _lab(r)
        c0 = pl.multiple_of(_ctile(l) * _TILE, _TILE)
        row0 = pl.multiple_of(base + (r // 8) * 8, 8)
        cp = pltpu.make_async_copy(
            cosine_hbm.at[pl.ds(row0, 8), pl.ds(c0, _TILE)],
            win_v.at[r],
            sem,
        )
        cp.start()
        copies.append(cp)
    for cp in copies:
        cp.wait()

    iot = lax.iota(jnp.int32, _WIN)
    for g in range(_RPW // 16):
        lv = labs[g]
        ct = jnp.minimum(jnp.maximum(lv >> 7, 0), _MAX_CTILE)
        off = jnp.minimum(lv - ct * _TILE, _TILE - 1)
        ridx = g * 16 + iot
        fix_v[pl.ds(g * 16, _WIN)] = plsc.load_gather(
            win_v, [ridx, ridx & 7, off]
        )
    pltpu.sync_copy(fix_v, fix_hbm.at[pl.ds(base, _RPW)])


_sc_gather = functools.partial(
    pl.kernel,
    mesh=plsc.VectorSubcoreMesh(core_axis_name="c", subcore_axis_name="s"),
    out_type=jax.ShapeDtypeStruct((_N_ROWS,), jnp.float32),
    compiler_params=pltpu.CompilerParams(needs_layout_passes=False),
    scratch_types=[
        pltpu.VMEM((_RPW,), jnp.int32),
        pltpu.VMEM((_RPW, 8, _TILE), jnp.float32),
        pltpu.VMEM((_RPW,), jnp.float32),
        pltpu.SemaphoreType.DMA,
    ],
)(_sc_gather_body)


def _dense_body(label_ref, fixv_ref, scale_ref, cosine_ref, out_ref):
    x = cosine_ref[...]
    s = scale_ref[0]
    lab = label_ref[0, 0, :]
    xg = fixv_ref[0, 0, :]
    fv = (xg * COS_M - jnp.sqrt(jnp.maximum(1.0 - xg * xg, 0.0)) * SIN_M) * s
    cols = jax.lax.broadcasted_iota(jnp.int32, x.shape, 1)
    hit = cols == lab[:, None]
    out_ref[...] = jnp.where(hit, fv[:, None], x * s)


@jax.jit
def kernel(cosine, label, scale):
    n_rows, n_cols = cosine.shape
    fixv = _sc_gather(cosine, label)
    n_blocks = n_rows // ROW_BLOCK
    lab3 = label.reshape(n_blocks, 1, ROW_BLOCK)
    fixv3 = fixv.reshape(n_blocks, 1, ROW_BLOCK)
    return pl.pallas_call(
        _dense_body,
        grid=(n_blocks,),
        in_specs=[
            pl.BlockSpec((1, 1, ROW_BLOCK), lambda i: (i, 0, 0)),
            pl.BlockSpec((1, 1, ROW_BLOCK), lambda i: (i, 0, 0)),
            pl.BlockSpec((1,), lambda i: (0,)),
            pl.BlockSpec((ROW_BLOCK, COL_BLOCK), lambda i: (i, 0)),
        ],
        out_specs=pl.BlockSpec((ROW_BLOCK, COL_BLOCK), lambda i: (i, 0)),
        out_shape=jax.ShapeDtypeStruct(cosine.shape, cosine.dtype),
    )(lab3, fixv3, scale, cosine)


# speedup vs baseline: 1.1025x; 1.0009x over previous
import functools
import math

import jax
import jax.numpy as jnp
from jax import lax
from jax.experimental import pallas as pl
from jax.experimental.pallas import tpu as pltpu
from jax.experimental.pallas import tpu_sc as plsc

M = 0.5
COS_M = math.cos(M)
SIN_M = math.sin(M)

ROW_BLOCK = 16
COL_BLOCK = 100000

_NC = 2
_NS = 16
_NW = _NC * _NS
_N_ROWS = 1024
_RPW = _N_ROWS // _NW
_WIN = 16
_N_COLS = 100000
_TILE = 128
_MAX_CTILE = (_N_COLS - 1) // _TILE


def _sc_gather_body(cosine_hbm, label_hbm, fix_hbm, lab_v, win_v, fix_v, sem):
    wid = lax.axis_index("s") * _NC + lax.axis_index("c")
    base = wid * _RPW
    pltpu.sync_copy(label_hbm.at[pl.ds(base, _RPW)], lab_v)

    labs = [lab_v[pl.ds(g * 16, 16)] for g in range(_RPW // 16)]

    def _lab(r):
        return labs[r // 16][r % 16]

    def _ctile(l):
        return jnp.minimum(jnp.maximum(l >> 7, 0), _MAX_CTILE)

    copies = []
    for r in range(_RPW):
        l = _lab(r)
        c0 = pl.multiple_of(_ctile(l) * _TILE, _TILE)
        row0 = pl.multiple_of(base + (r // 8) * 8, 8)
        cp = pltpu.make_async_copy(
            cosine_hbm.at[pl.ds(row0, 8), pl.ds(c0, _TILE)],
            win_v.at[r],
            sem,
        )
        cp.start()
        copies.append(cp)
    for cp in copies:
        cp.wait()

    iot = lax.iota(jnp.int32, _WIN)
    for g in range(_RPW // 16):
        lv = labs[g]
        ct = jnp.minimum(jnp.maximum(lv >> 7, 0), _MAX_CTILE)
        off = jnp.minimum(jnp.maximum(lv - ct * _TILE, 0), _TILE - 1)
        ridx = g * 16 + iot
        fix_v[pl.ds(g * 16, _WIN)] = plsc.load_gather(
            win_v, [ridx, ridx & 7, off]
        )
    pltpu.sync_copy(fix_v, fix_hbm.at[pl.ds(base, _RPW)])


_sc_gather = functools.partial(
    pl.kernel,
    mesh=plsc.VectorSubcoreMesh(core_axis_name="c", subcore_axis_name="s"),
    out_type=jax.ShapeDtypeStruct((_N_ROWS,), jnp.float32),
    compiler_params=pltpu.CompilerParams(needs_layout_passes=False),
    scratch_types=[
        pltpu.VMEM((_RPW,), jnp.int32),
        pltpu.VMEM((_RPW, 8, _TILE), jnp.float32),
        pltpu.VMEM((_RPW,), jnp.float32),
        pltpu.SemaphoreType.DMA,
    ],
)(_sc_gather_body)


def _dense_body(label_ref, fixv_ref, scale_ref, cosine_ref, out_ref):
    x = cosine_ref[...]
    s = scale_ref[0]
    lab = label_ref[0, 0, :]
    xg = fixv_ref[0, 0, :]
    fv = (xg * COS_M - jnp.sqrt(jnp.maximum(1.0 - xg * xg, 0.0)) * SIN_M) * s
    cols = jax.lax.broadcasted_iota(jnp.int32, x.shape, 1)
    hit = cols == lab[:, None]
    out_ref[...] = jnp.where(hit, fv[:, None], x * s)


@jax.jit
def kernel(cosine, label, scale):
    n_rows, n_cols = cosine.shape
    fixv = _sc_gather(cosine, label)
    n_blocks = n_rows // ROW_BLOCK
    lab3 = label.reshape(n_blocks, 1, ROW_BLOCK)
    fixv3 = fixv.reshape(n_blocks, 1, ROW_BLOCK)
    return pl.pallas_call(
        _dense_body,
        grid=(n_blocks,),
        in_specs=[
            pl.BlockSpec((1, 1, ROW_BLOCK), lambda i: (i, 0, 0)),
            pl.BlockSpec((1, 1, ROW_BLOCK), lambda i: (i, 0, 0)),
            pl.BlockSpec((1,), lambda i: (0,)),
            pl.BlockSpec((ROW_BLOCK, COL_BLOCK), lambda i: (i, 0)),
        ],
        out_specs=pl.BlockSpec((ROW_BLOCK, COL_BLOCK), lambda i: (i, 0)),
        out_shape=jax.ShapeDtypeStruct(cosine.shape, cosine.dtype),
    )(lab3, fixv3, scale, cosine)


# speedup vs baseline: 3.9299x; 3.5644x over previous
import functools
import math

import jax
import jax.numpy as jnp
from jax import lax
from jax.experimental import pallas as pl
from jax.experimental.pallas import tpu as pltpu
from jax.experimental.pallas import tpu_sc as plsc

M = 0.5
COS_M = math.cos(M)
SIN_M = math.sin(M)

CLS_BLOCK = 2048

_NC = 2
_NS = 16
_NW = _NC * _NS
_N_ROWS = 1024
_RPW = _N_ROWS // _NW
_WIN = 16
_TILE = 128


def _sc_gather_body(cosinet_hbm, label_hbm, fix_hbm, lab_v, win_v, fix_v, sem):
    wid = lax.axis_index("s") * _NC + lax.axis_index("c")
    base = wid * _RPW
    pltpu.sync_copy(label_hbm.at[pl.ds(base, _RPW)], lab_v)

    labs = [lab_v[pl.ds(g * 16, 16)] for g in range(_RPW // 16)]

    def _lab(r):
        return labs[r // 16][r % 16]

    cbase = pl.multiple_of((base >> 7) << 7, _TILE)

    copies = []
    for r in range(_RPW):
        l = jnp.maximum(_lab(r), 0)
        row0 = pl.multiple_of((l >> 3) << 3, 8)
        cp = pltpu.make_async_copy(
            cosinet_hbm.at[pl.ds(row0, 8), pl.ds(cbase, _TILE)],
            win_v.at[r],
            sem,
        )
        cp.start()
        copies.append(cp)
    for cp in copies:
        cp.wait()

    iot = lax.iota(jnp.int32, _WIN)
    base127 = base & (_TILE - 1)
    for g in range(_RPW // 16):
        lv = jnp.maximum(labs[g], 0)
        ridx = g * 16 + iot
        fix_v[pl.ds(g * 16, _WIN)] = plsc.load_gather(
            win_v, [ridx, lv & 7, base127 + ridx]
        )
    pltpu.sync_copy(fix_v, fix_hbm.at[pl.ds(base, _RPW)])


_sc_gather = functools.partial(
    pl.kernel,
    mesh=plsc.VectorSubcoreMesh(core_axis_name="c", subcore_axis_name="s"),
    out_type=jax.ShapeDtypeStruct((_N_ROWS,), jnp.float32),
    compiler_params=pltpu.CompilerParams(needs_layout_passes=False),
    scratch_types=[
        pltpu.VMEM((_RPW,), jnp.int32),
        pltpu.VMEM((_RPW, 8, _TILE), jnp.float32),
        pltpu.VMEM((_RPW,), jnp.float32),
        pltpu.SemaphoreType.DMA,
    ],
)(_sc_gather_body)


def _dense_body(label_ref, fixv_ref, scale_ref, cosinet_ref, out_ref):
    i = pl.program_id(0)
    x = cosinet_ref[...]
    s = scale_ref[0]
    lab = label_ref[...]
    xg = fixv_ref[...]
    fv = (xg * COS_M - jnp.sqrt(jnp.maximum(1.0 - xg * xg, 0.0)) * SIN_M) * s
    cls = i * CLS_BLOCK + jax.lax.broadcasted_iota(jnp.int32, x.shape, 0)
    hit = cls == lab
    out_ref[...] = jnp.where(hit, fv, x * s)


@jax.jit
def kernel(cosine, label, scale):
    n_rows, n_cls = cosine.shape
    ct = cosine.T
    fixv = _sc_gather(ct, label)
    lab2 = label.reshape(1, n_rows)
    fixv2 = fixv.reshape(1, n_rows)
    out_t = pl.pallas_call(
        _dense_body,
        grid=(pl.cdiv(n_cls, CLS_BLOCK),),
        in_specs=[
            pl.BlockSpec((1, n_rows), lambda i: (0, 0)),
            pl.BlockSpec((1, n_rows), lambda i: (0, 0)),
            pl.BlockSpec((1,), lambda i: (0,)),
            pl.BlockSpec((CLS_BLOCK, n_rows), lambda i: (i, 0)),
        ],
        out_specs=pl.BlockSpec((CLS_BLOCK, n_rows), lambda i: (i, 0)),
        out_shape=jax.ShapeDtypeStruct((n_cls, n_rows), cosine.dtype),
    )(lab2, fixv2, scale, ct)
    return out_t.T


# speedup vs baseline: 3.9335x; 1.0009x over previous
import functools
import math

import jax
import jax.numpy as jnp
from jax import lax
from jax.experimental import pallas as pl
from jax.experimental.pallas import tpu as pltpu
from jax.experimental.pallas import tpu_sc as plsc

M = 0.5
COS_M = math.cos(M)
SIN_M = math.sin(M)

CLS_BLOCK = 2560

_NC = 2
_NS = 16
_NW = _NC * _NS
_N_ROWS = 1024
_RPW = _N_ROWS // _NW
_WIN = 16
_TILE = 128


def _sc_gather_body(cosinet_hbm, label_hbm, fix_hbm, lab_v, win_v, fix_v, sem):
    wid = lax.axis_index("s") * _NC + lax.axis_index("c")
    base = wid * _RPW
    pltpu.sync_copy(label_hbm.at[pl.ds(base, _RPW)], lab_v)

    labs = [lab_v[pl.ds(g * 16, 16)] for g in range(_RPW // 16)]

    def _lab(r):
        return labs[r // 16][r % 16]

    cbase = pl.multiple_of((base >> 7) << 7, _TILE)

    copies = []
    for r in range(_RPW):
        l = jnp.maximum(_lab(r), 0)
        row0 = pl.multiple_of((l >> 3) << 3, 8)
        cp = pltpu.make_async_copy(
            cosinet_hbm.at[pl.ds(row0, 8), pl.ds(cbase, _TILE)],
            win_v.at[r],
            sem,
        )
        cp.start()
        copies.append(cp)
    for cp in copies:
        cp.wait()

    iot = lax.iota(jnp.int32, _WIN)
    base127 = base & (_TILE - 1)
    for g in range(_RPW // 16):
        lv = jnp.maximum(labs[g], 0)
        ridx = g * 16 + iot
        fix_v[pl.ds(g * 16, _WIN)] = plsc.load_gather(
            win_v, [ridx, lv & 7, base127 + ridx]
        )
    pltpu.sync_copy(fix_v, fix_hbm.at[pl.ds(base, _RPW)])


_sc_gather = functools.partial(
    pl.kernel,
    mesh=plsc.VectorSubcoreMesh(core_axis_name="c", subcore_axis_name="s"),
    out_type=jax.ShapeDtypeStruct((_N_ROWS,), jnp.float32),
    compiler_params=pltpu.CompilerParams(needs_layout_passes=False),
    scratch_types=[
        pltpu.VMEM((_RPW,), jnp.int32),
        pltpu.VMEM((_RPW, 8, _TILE), jnp.float32),
        pltpu.VMEM((_RPW,), jnp.float32),
        pltpu.SemaphoreType.DMA,
    ],
)(_sc_gather_body)


def _dense_body(label_ref, fixv_ref, scale_ref, cosinet_ref, out_ref):
    i = pl.program_id(0)
    x = cosinet_ref[...]
    s = scale_ref[0]
    lab = label_ref[...]
    xg = fixv_ref[...]
    fv = (xg * COS_M - jnp.sqrt(jnp.maximum(1.0 - xg * xg, 0.0)) * SIN_M) * s
    cls = i * CLS_BLOCK + jax.lax.broadcasted_iota(jnp.int32, x.shape, 0)
    hit = cls == lab
    out_ref[...] = jnp.where(hit, fv, x * s)


@jax.jit
def kernel(cosine, label, scale):
    n_rows, n_cls = cosine.shape
    ct = cosine.T
    fixv = _sc_gather(ct, label)
    lab2 = label.reshape(1, n_rows)
    fixv2 = fixv.reshape(1, n_rows)
    out_t = pl.pallas_call(
        _dense_body,
        grid=(pl.cdiv(n_cls, CLS_BLOCK),),
        in_specs=[
            pl.BlockSpec((1, n_rows), lambda i: (0, 0)),
            pl.BlockSpec((1, n_rows), lambda i: (0, 0)),
            pl.BlockSpec((1,), lambda i: (0,)),
            pl.BlockSpec((CLS_BLOCK, n_rows), lambda i: (i, 0)),
        ],
        out_specs=pl.BlockSpec((CLS_BLOCK, n_rows), lambda i: (i, 0)),
        out_shape=jax.ShapeDtypeStruct((n_cls, n_rows), cosine.dtype),
    )(lab2, fixv2, scale, ct)
    return out_t.T
